# Initial kernel scaffold; baseline (speedup 1.0000x reference)
import jax
import jax.numpy as jnp
from jax.experimental import pallas as pl


def kernel(p_feats, v_feats, p2v_idx, Wp, bp, Wv, bv):
    raise NotImplementedError("write your pallas kernel here")


# speedup vs baseline: 2.6544x; 2.6544x over previous
import functools

import jax
import jax.numpy as jnp
from jax import lax
from jax.experimental import pallas as pl
from jax.experimental.pallas import tpu as pltpu
from jax.experimental.pallas import tpu_sc as plsc

B = 262144
V = 65536
C = 64
NC = 2
NS = 16
NW = NC * NS
CH = 128
NCHUNKS = B // CH
SHARD = 16384
NSHARD = V // SHARD
SROWS = SHARD
ZROWS = SROWS // NS

_mesh = plsc.VectorSubcoreMesh(core_axis_name="c", subcore_axis_name="s")
_sc_params = pltpu.CompilerParams(use_tc_tiling_on_sc=False, needs_layout_passes=False)


def _gather_body(v_hbm, idx_hbm, out_hbm, idx_v, rows_v, gsem):
    cid = lax.axis_index("c")
    sid = lax.axis_index("s")
    wid = sid * NC + cid
    nch = NCHUNKS // NW
    ch0 = wid * nch
    pltpu.sync_copy(idx_hbm.at[pl.ds(ch0, nch)], idx_v)

    nbuf = 4
    for b in range(nbuf):
        pltpu.async_copy(v_hbm.at[idx_v.at[b]], rows_v.at[b], gsem)

    def body(g, carry):
        for b in range(nbuf):
            j = g * nbuf + b
            pltpu.make_async_copy(
                v_hbm.at[idx_v.at[j]], rows_v.at[b], gsem).wait()
            pltpu.sync_copy(rows_v.at[b],
                            out_hbm.at[pl.ds((ch0 + j) * CH, CH)])

            @pl.when(j + nbuf < nch)
            def _():
                pltpu.async_copy(v_hbm.at[idx_v.at[j + nbuf]],
                                 rows_v.at[b], gsem)
        return carry

    lax.fori_loop(0, nch // nbuf, body, 0)


@functools.partial(jax.jit, donate_argnums=())
def _gather(v_feats, idx2d):
    k = pl.kernel(
        _gather_body,
        out_type=jax.ShapeDtypeStruct((B, C), jnp.float32),
        mesh=_mesh,
        scratch_types=[
            pltpu.VMEM((NCHUNKS // NW, CH), jnp.int32),
            pltpu.VMEM((4, CH, C), jnp.float32),
            pltpu.SemaphoreType.DMA,
        ],
        compiler_params=_sc_params,
    )
    return k(v_feats, idx2d)


_BLK = 2048
_IB = _BLK // CH


def _fuse_body(p_ref, v_ref, idx_ref, wpd_ref, wvd_ref, bias_ref,
               fuse_ref, loc_ref):
    p = p_ref[...]
    v = v_ref[...]
    d = (jnp.sum(p * wpd_ref[...], axis=1, keepdims=True)
         + jnp.sum(v * wvd_ref[...], axis=1, keepdims=True)
         + bias_ref[0, 0])
    s = 1.0 / (1.0 + jnp.exp(-d))
    fuse_ref[...] = v + s * (p - v)

    idx = idx_ref[...]
    for sh in range(NSHARD):
        loc = idx - sh * SHARD
        ok = (loc >= 0) & (loc < SHARD)
        loc_ref[sh, :, :] = jnp.where(ok, loc, -1)


def _fuse_tc(p_feats, v2p, idx2d, wpd, wvd, bias):
    grid = (B // _BLK,)
    return pl.pallas_call(
        _fuse_body,
        grid=grid,
        in_specs=[
            pl.BlockSpec((_BLK, C), lambda i: (i, 0)),
            pl.BlockSpec((_BLK, C), lambda i: (i, 0)),
            pl.BlockSpec((_IB, CH), lambda i: (i, 0)),
            pl.BlockSpec((1, C), lambda i: (0, 0)),
            pl.BlockSpec((1, C), lambda i: (0, 0)),
            pl.BlockSpec((1, 1), lambda i: (0, 0)),
        ],
        out_specs=[
            pl.BlockSpec((_BLK, C), lambda i: (i, 0)),
            pl.BlockSpec((NSHARD, _IB, CH), lambda i: (0, i, 0)),
        ],
        out_shape=[
            jax.ShapeDtypeStruct((B, C), jnp.float32),
            jax.ShapeDtypeStruct((NSHARD, NCHUNKS, CH), jnp.int32),
        ],
    )(p_feats, v2p, idx2d, wpd, wvd, bias)


def _scatter_body(fuse_hbm, loc_hbm, vnew_hbm,
                  sums_sh, cnts_sh,
                  loc_all, fuse_buf, ones_buf, zero_buf, zcnt_buf, fsem):
    cid = lax.axis_index("c")
    sid = lax.axis_index("s")
    cpt = NCHUNKS // NS

    zero16 = jnp.zeros((16,), jnp.float32)
    one16 = jnp.full((16,), 1.0, jnp.float32)
    for r in range(CH):
        zcnt_buf[r, :] = zero16
        for c4 in range(C // 16):
            zero_buf[r, pl.ds(c4 * 16, 16)] = zero16

    for ps in range(NSHARD // NC):
        shard_id = cid * (NSHARD // NC) + ps
        shard_base = shard_id * SHARD

        for r in range(CH):
            ones_buf[r, :] = one16

        z0 = sid * ZROWS
        for z in range(ZROWS // CH):
            pltpu.sync_copy(zero_buf, sums_sh.at[pl.ds(z0 + z * CH, CH)])
            pltpu.sync_copy(zcnt_buf, cnts_sh.at[pl.ds(z0 + z * CH, CH)])
        plsc.subcore_barrier()

        pltpu.sync_copy(loc_hbm.at[shard_id, pl.ds(sid * cpt, cpt)], loc_all)

        nbuf = 2
        for b in range(nbuf):
            pltpu.async_copy(
                fuse_hbm.at[pl.ds((sid * cpt + b) * CH, CH)],
                fuse_buf.at[b], fsem)

        def body(g, carry):
            for b in range(nbuf):
                j = g * nbuf + b
                pltpu.make_async_copy(
                    fuse_hbm.at[pl.ds((sid * cpt + j) * CH, CH)],
                    fuse_buf.at[b], fsem).wait()
                locs = plsc.Indices(loc_all.at[j], ignored_value=-1)
                pltpu.sync_copy(fuse_buf.at[b], sums_sh.at[locs], add=True)
                pltpu.sync_copy(ones_buf, cnts_sh.at[locs], add=True)

                @pl.when(j + nbuf < cpt)
                def _():
                    pltpu.async_copy(
                        fuse_hbm.at[pl.ds((sid * cpt + j + nbuf) * CH, CH)],
                        fuse_buf.at[b], fsem)
            return carry

        lax.fori_loop(0, cpt // nbuf, body, 0)
        plsc.subcore_barrier()

        r0 = sid * (SHARD // NS)

        def wbody(g, carry):
            row = r0 + g * CH
            pltpu.sync_copy(sums_sh.at[pl.ds(row, CH)], fuse_buf.at[0])
            pltpu.sync_copy(cnts_sh.at[pl.ds(row, CH)], ones_buf)
            for r in range(CH):
                cvec = ones_buf[r, :]
                inv = 1.0 / jnp.maximum(cvec, 1.0)
                for c4 in range(C // 16):
                    fuse_buf[1, r, pl.ds(c4 * 16, 16)] = (
                        fuse_buf[0, r, pl.ds(c4 * 16, 16)] * inv)
            pltpu.sync_copy(fuse_buf.at[1],
                            vnew_hbm.at[pl.ds(shard_base + row, CH)])
            return carry

        lax.fori_loop(0, SHARD // NS // CH, wbody, 0)
        plsc.subcore_barrier()


def _scatter(fuse, loc4):
    k = pl.kernel(
        _scatter_body,
        out_type=jax.ShapeDtypeStruct((V, C), jnp.float32),
        mesh=_mesh,
        scratch_types=[
            pltpu.VMEM_SHARED((SROWS, C), jnp.float32),
            pltpu.VMEM_SHARED((SROWS, 16), jnp.float32),
            pltpu.VMEM((NCHUNKS // NS, CH), jnp.int32),
            pltpu.VMEM((2, CH, C), jnp.float32),
            pltpu.VMEM((CH, 16), jnp.float32),
            pltpu.VMEM((CH, C), jnp.float32),
            pltpu.VMEM((CH, 16), jnp.float32),
            pltpu.SemaphoreType.DMA,
        ],
        compiler_params=_sc_params,
    )
    return k(fuse, loc4)


def kernel(p_feats, v_feats, p2v_idx, Wp, bp, Wv, bv):
    idx2d = p2v_idx.reshape(NCHUNKS, CH)
    wpd = (Wp[0] - Wp[1]).reshape(1, C)
    wvd = (Wv[0] - Wv[1]).reshape(1, C)
    bias = (bp[0] - bp[1] + bv[0] - bv[1]).reshape(1, 1)
    v2p = _gather(v_feats, idx2d)
    fuse, loc4 = _fuse_tc(p_feats, v2p, idx2d, wpd, wvd, bias)
    v_new = _scatter(fuse, loc4)
    return fuse, v_new


# speedup vs baseline: 3.8383x; 1.4460x over previous
import functools

import jax
import jax.numpy as jnp
from jax import lax
from jax.experimental import pallas as pl
from jax.experimental.pallas import tpu as pltpu
from jax.experimental.pallas import tpu_sc as plsc

B = 262144
V = 65536
C = 64
NC = 2
NS = 16
NW = NC * NS
L = 128
BT = B // L
VT = V // L
CG = C // 8

_mesh = plsc.VectorSubcoreMesh(core_axis_name="c", subcore_axis_name="s")
_sc_params = pltpu.CompilerParams(use_tc_tiling_on_sc=False,
                                  needs_layout_passes=False)


def _stage_idx(idx_hbm, idx_sp, sid):
    pltpu.sync_copy(idx_hbm.at[pl.ds(sid * (BT // NS), BT // NS)],
                    idx_sp.at[pl.ds(sid * (BT // NS), BT // NS)])
    plsc.subcore_barrier()


def _gather_body(v4d, idx_hbm, out4d, idx_sp, tab_v, idxb, outb, gsem):
    cid = lax.axis_index("c")
    sid = lax.axis_index("s")
    wid = sid * NC + cid
    _stage_idx(idx_hbm, idx_sp, sid)

    for c2 in range(2):
        ch = wid * 2 + c2
        g = ch // 8
        r = ch % 8
        pltpu.sync_copy(v4d.at[g, pl.ds(0, VT), r], tab_v)

        nbuf = 2
        def chunk(j, b):
            pltpu.sync_copy(idx_sp.at[pl.ds(j * 16, 16)], idxb)
            for rr in range(16):
                for k in range(8):
                    iv = idxb[rr, pl.ds(k * 16, 16)]
                    hi = lax.shift_right_logical(iv, 7)
                    lo = lax.bitwise_and(iv, 127)
                    outb[b, rr, pl.ds(k * 16, 16)] = (
                        plsc.load_gather(tab_v, [hi, lo]))
            pltpu.async_copy(outb.at[b], out4d.at[g, pl.ds(j * 16, 16), r],
                             gsem)

        def body(gi, carry):
            for b2 in range(nbuf):
                j = gi * nbuf + b2
                @pl.when(j >= nbuf)
                def _():
                    pltpu.make_async_copy(
                        outb.at[b2],
                        out4d.at[g, pl.ds((j - nbuf) * 16, 16), r],
                        gsem).wait()
                chunk(j, b2)
            return carry

        lax.fori_loop(0, (BT // 16) // nbuf, body, 0)
        for b2 in range(nbuf):
            pltpu.make_async_copy(
                outb.at[b2],
                out4d.at[g, pl.ds(0, 16), r], gsem).wait()


def _gather_cm(v4d, idx2d):
    k = pl.kernel(
        _gather_body,
        out_type=jax.ShapeDtypeStruct((CG, BT, 8, L), jnp.float32),
        mesh=_mesh,
        scratch_types=[
            pltpu.VMEM_SHARED((BT, L), jnp.int32),
            pltpu.VMEM((VT, L), jnp.float32),
            pltpu.VMEM((16, L), jnp.int32),
            pltpu.VMEM((2, 16, L), jnp.float32),
            pltpu.SemaphoreType.DMA,
        ],
        compiler_params=_sc_params,
    )
    return k(v4d, idx2d)


_BLKP = 4096


def _fuse_body(p_ref, v_ref, wpd_ref, wvd_ref, bias_ref, fuse_ref):
    p = p_ref[...]
    v = v_ref[...]
    wpd = wpd_ref[...][:, :1]
    wvd = wvd_ref[...][:, :1]
    d = (jnp.sum(p * wpd, axis=0, keepdims=True)
         + jnp.sum(v * wvd, axis=0, keepdims=True)
         + bias_ref[0, 0])
    s = 1.0 / (1.0 + jnp.exp(-d))
    fuse_ref[...] = v + s * (p - v)


def _fuse_tc(pT, v2pT, wpd, wvd, bias):
    return pl.pallas_call(
        _fuse_body,
        grid=(B // _BLKP,),
        in_specs=[
            pl.BlockSpec((C, _BLKP), lambda i: (0, i)),
            pl.BlockSpec((C, _BLKP), lambda i: (0, i)),
            pl.BlockSpec((C, 128), lambda i: (0, 0)),
            pl.BlockSpec((C, 128), lambda i: (0, 0)),
            pl.BlockSpec((1, 1), lambda i: (0, 0)),
        ],
        out_specs=pl.BlockSpec((C, _BLKP), lambda i: (0, i)),
        out_shape=jax.ShapeDtypeStruct((C, B), jnp.float32),
    )(pT, v2pT, wpd, wvd, bias)


def _scatter_body(f4d, idx_hbm, out4d,
                  idx_sp, cnt_sp, inv_sp, zsp,
                  bins, idxb, fuseb, invb, outb, zbuf, iden, ssem):
    cid = lax.axis_index("c")
    sid = lax.axis_index("s")
    wid = sid * NC + cid
    zero16 = jnp.zeros((16,), jnp.float32)
    one16 = jnp.full((16,), 1.0, jnp.float32)

    for rr in range(16):
        for k in range(8):
            zbuf[rr, pl.ds(k * 16, 16)] = zero16
    for q in range(4):
        for k in range(8):
            iden[q, pl.ds(k * 16, 16)] = (
                lax.iota(jnp.int32, 16) + (q * L + k * 16))

    @pl.when(sid == 0)
    def _():
        pltpu.sync_copy(zbuf, zsp)

    _stage_idx(idx_hbm, idx_sp, sid)

    def zero_bins():
        for z in range(VT // 16):
            pltpu.sync_copy(zsp, bins.at[pl.ds(z * 16, 16)])

    zero_bins()
    for z in range(VT // NS // 16):
        pltpu.sync_copy(zbuf, cnt_sp.at[pl.ds(sid * (VT // NS) + z * 16, 16)])
    plsc.subcore_barrier()

    def cbody(j, carry):
        pltpu.sync_copy(idx_sp.at[pl.ds(sid * (BT // NS) + j * 4, 4)],
                        idxb.at[pl.ds(0, 4)])
        for rr in range(4):
            for k in range(8):
                iv = idxb[rr, pl.ds(k * 16, 16)]
                hi = lax.shift_right_logical(iv, 7)
                lo = lax.bitwise_and(iv, 127)
                plsc.addupdate_scatter(bins, [hi, lo], one16)
        return carry

    lax.fori_loop(0, BT // NS // 4, cbody, 0)
    for q in range(4):
        pltpu.sync_copy(bins.at[pl.ds(q * L, L)],
                        cnt_sp.at[plsc.Indices(iden.at[q])], add=True)
    plsc.subcore_barrier()

    s0 = sid * (VT // NS)
    for z in range(VT // NS // 16):
        pltpu.sync_copy(cnt_sp.at[pl.ds(s0 + z * 16, 16)], invb)
        for rr in range(16):
            for k in range(8):
                cv = invb[rr, pl.ds(k * 16, 16)]
                outb[rr, pl.ds(k * 16, 16)] = 1.0 / jnp.maximum(cv, 1.0)
        pltpu.sync_copy(outb, inv_sp.at[pl.ds(s0 + z * 16, 16)])
    plsc.subcore_barrier()

    for c2 in range(2):
        ch = wid * 2 + c2
        g = ch // 8
        r = ch % 8
        zero_bins()

        def sbody(gi, carry):
            for b2 in range(2):
                j = gi * 2 + b2
                pltpu.make_async_copy(
                    f4d.at[g, pl.ds(j * 16, 16), r], fuseb.at[b2],
                    ssem).wait()
                pltpu.sync_copy(idx_sp.at[pl.ds(j * 16, 16)], idxb)
                for rr in range(16):
                    for k in range(8):
                        iv = idxb[rr, pl.ds(k * 16, 16)]
                        hi = lax.shift_right_logical(iv, 7)
                        lo = lax.bitwise_and(iv, 127)
                        plsc.addupdate_scatter(
                            bins, [hi, lo], fuseb[b2, rr, pl.ds(k * 16, 16)])

                @pl.when(j + 2 < BT // 16)
                def _():
                    pltpu.async_copy(
                        f4d.at[g, pl.ds((j + 2) * 16, 16), r],
                        fuseb.at[b2], ssem)
            return carry

        for b2 in range(2):
            pltpu.async_copy(f4d.at[g, pl.ds(b2 * 16, 16), r],
                             fuseb.at[b2], ssem)
        lax.fori_loop(0, (BT // 16) // 2, sbody, 0)

        def wbody(w, carry):
            pltpu.sync_copy(inv_sp.at[pl.ds(w * 16, 16)], invb)
            for rr in range(16):
                for k in range(8):
                    outb[rr, pl.ds(k * 16, 16)] = (
                        bins[w * 16 + rr, pl.ds(k * 16, 16)]
                        * invb[rr, pl.ds(k * 16, 16)])
            pltpu.sync_copy(outb, out4d.at[g, pl.ds(w * 16, 16), r])
            return carry

        lax.fori_loop(0, VT // 16, wbody, 0)


def _scatter_cm(f4d, idx2d):
    k = pl.kernel(
        _scatter_body,
        out_type=jax.ShapeDtypeStruct((CG, VT, 8, L), jnp.float32),
        mesh=_mesh,
        scratch_types=[
            pltpu.VMEM_SHARED((BT, L), jnp.int32),
            pltpu.VMEM_SHARED((VT, L), jnp.float32),
            pltpu.VMEM_SHARED((VT, L), jnp.float32),
            pltpu.VMEM_SHARED((16, L), jnp.float32),
            pltpu.VMEM((VT, L), jnp.float32),
            pltpu.VMEM((16, L), jnp.int32),
            pltpu.VMEM((2, 16, L), jnp.float32),
            pltpu.VMEM((16, L), jnp.float32),
            pltpu.VMEM((16, L), jnp.float32),
            pltpu.VMEM((16, L), jnp.float32),
            pltpu.VMEM((4, L), jnp.int32),
            pltpu.SemaphoreType.DMA,
        ],
        compiler_params=_sc_params,
    )
    return k(f4d, idx2d)


def kernel(p_feats, v_feats, p2v_idx, Wp, bp, Wv, bv):
    idx2d = p2v_idx.reshape(BT, L)
    wpd = jnp.broadcast_to((Wp[0] - Wp[1]).reshape(C, 1), (C, 128))
    wvd = jnp.broadcast_to((Wv[0] - Wv[1]).reshape(C, 1), (C, 128))
    bias = (bp[0] - bp[1] + bv[0] - bv[1]).reshape(1, 1)

    v4d = v_feats.reshape(VT, L, CG, 8).transpose(2, 0, 3, 1)
    pT = p_feats.T

    v2p4d = _gather_cm(v4d, idx2d)
    v2pT = v2p4d.transpose(0, 2, 1, 3).reshape(C, B)
    fuseT = _fuse_tc(pT, v2pT, wpd, wvd, bias)
    f4d = fuseT.reshape(CG, 8, BT, L).transpose(0, 2, 1, 3)
    vnew4d = _scatter_cm(f4d, idx2d)
    v_new = vnew4d.transpose(0, 2, 1, 3).reshape(C, V).T
    return fuseT.T, v_new


# speedup vs baseline: 5.6403x; 1.4695x over previous
import functools

import jax
import jax.numpy as jnp
from jax import lax
from jax.experimental import pallas as pl
from jax.experimental.pallas import tpu as pltpu
from jax.experimental.pallas import tpu_sc as plsc

B = 262144
V = 65536
C = 64
NC = 2
NS = 16
NW = NC * NS
L = 128
BT = B // L
VT = V // L
CG = C // 8

_mesh = plsc.VectorSubcoreMesh(core_axis_name="c", subcore_axis_name="s")
_sc_params = pltpu.CompilerParams(use_tc_tiling_on_sc=False,
                                  needs_layout_passes=False)


def _stage_idx(idx_hbm, idx_sp, sid):
    pltpu.sync_copy(idx_hbm.at[pl.ds(sid * (BT // NS), BT // NS)],
                    idx_sp.at[pl.ds(sid * (BT // NS), BT // NS)])
    plsc.subcore_barrier()


def _gather_body(v4d, idx_hbm, out4d, idx_sp, tab_v, idxb, outb, gsem):
    cid = lax.axis_index("c")
    sid = lax.axis_index("s")
    wid = sid * NC + cid
    _stage_idx(idx_hbm, idx_sp, sid)

    for c2 in range(2):
        ch = wid * 2 + c2
        g = ch // 8
        r = ch % 8
        pltpu.sync_copy(v4d.at[g, pl.ds(0, VT), r], tab_v)

        nbuf = 2
        def chunk(j, b):
            pltpu.sync_copy(idx_sp.at[pl.ds(j * 16, 16)], idxb)

            @plsc.parallel_loop(0, 16, unroll=4)
            def _(rr):
                for k in range(8):
                    iv = idxb[rr, pl.ds(k * 16, 16)]
                    hi = lax.shift_right_logical(iv, 7)
                    lo = lax.bitwise_and(iv, 127)
                    outb[b, rr, pl.ds(k * 16, 16)] = (
                        plsc.load_gather(tab_v, [hi, lo]))

            pltpu.async_copy(outb.at[b], out4d.at[g, pl.ds(j * 16, 16), r],
                             gsem)

        def body(gi, carry):
            for b2 in range(nbuf):
                j = gi * nbuf + b2
                @pl.when(j >= nbuf)
                def _():
                    pltpu.make_async_copy(
                        outb.at[b2],
                        out4d.at[g, pl.ds((j - nbuf) * 16, 16), r],
                        gsem).wait()
                chunk(j, b2)
            return carry

        lax.fori_loop(0, (BT // 16) // nbuf, body, 0)
        for b2 in range(nbuf):
            pltpu.make_async_copy(
                outb.at[b2],
                out4d.at[g, pl.ds(0, 16), r], gsem).wait()


def _gather_cm(v4d, idx2d):
    k = pl.kernel(
        _gather_body,
        out_type=jax.ShapeDtypeStruct((CG, BT, 8, L), jnp.float32),
        mesh=_mesh,
        scratch_types=[
            pltpu.VMEM_SHARED((BT, L), jnp.int32),
            pltpu.VMEM((VT, L), jnp.float32),
            pltpu.VMEM((16, L), jnp.int32),
            pltpu.VMEM((2, 16, L), jnp.float32),
            pltpu.SemaphoreType.DMA,
        ],
        compiler_params=_sc_params,
    )
    return k(v4d, idx2d)


_BLKP = 4096


def _fuse_body(p_ref, v_ref, wpd_ref, wvd_ref, bias_ref, fuse_ref):
    p = p_ref[...]
    v = v_ref[...]
    wpd = wpd_ref[...][:, :1]
    wvd = wvd_ref[...][:, :1]
    d = (jnp.sum(p * wpd, axis=0, keepdims=True)
         + jnp.sum(v * wvd, axis=0, keepdims=True)
         + bias_ref[0, 0])
    s = 1.0 / (1.0 + jnp.exp(-d))
    fuse_ref[...] = v + s * (p - v)


def _fuse_tc(pT, v2pT, wpd, wvd, bias):
    return pl.pallas_call(
        _fuse_body,
        grid=(B // _BLKP,),
        in_specs=[
            pl.BlockSpec((C, _BLKP), lambda i: (0, i)),
            pl.BlockSpec((C, _BLKP), lambda i: (0, i)),
            pl.BlockSpec((C, 128), lambda i: (0, 0)),
            pl.BlockSpec((C, 128), lambda i: (0, 0)),
            pl.BlockSpec((1, 1), lambda i: (0, 0)),
        ],
        out_specs=pl.BlockSpec((C, _BLKP), lambda i: (0, i)),
        out_shape=jax.ShapeDtypeStruct((C, B), jnp.float32),
    )(pT, v2pT, wpd, wvd, bias)


def _scatter_body(f4d, idx_hbm, out4d,
                  idx_sp, cnt_sp, inv_sp, zsp,
                  bins, idxb, fuseb, invb, outb, zbuf, iden, ssem):
    cid = lax.axis_index("c")
    sid = lax.axis_index("s")
    wid = sid * NC + cid
    zero16 = jnp.zeros((16,), jnp.float32)
    one16 = jnp.full((16,), 1.0, jnp.float32)

    for rr in range(16):
        for k in range(8):
            zbuf[rr, pl.ds(k * 16, 16)] = zero16
    for q in range(4):
        for k in range(8):
            iden[q, pl.ds(k * 16, 16)] = (
                lax.iota(jnp.int32, 16) + (q * L + k * 16))

    @pl.when(sid == 0)
    def _():
        pltpu.sync_copy(zbuf, zsp)

    _stage_idx(idx_hbm, idx_sp, sid)

    def zero_bins():
        for z in range(VT // 16):
            pltpu.sync_copy(zsp, bins.at[pl.ds(z * 16, 16)])

    zero_bins()
    for z in range(VT // NS // 16):
        pltpu.sync_copy(zbuf, cnt_sp.at[pl.ds(sid * (VT // NS) + z * 16, 16)])
    plsc.subcore_barrier()

    def cbody(j, carry):
        pltpu.sync_copy(idx_sp.at[pl.ds(sid * (BT // NS) + j * 4, 4)],
                        idxb.at[pl.ds(0, 4)])

        @plsc.parallel_loop(0, 4, unroll=2)
        def _(rr):
            for k in range(8):
                iv = idxb[rr, pl.ds(k * 16, 16)]
                hi = lax.shift_right_logical(iv, 7)
                lo = lax.bitwise_and(iv, 127)
                plsc.addupdate_scatter(bins, [hi, lo], one16)

        return carry

    lax.fori_loop(0, BT // NS // 4, cbody, 0)
    for q in range(4):
        pltpu.sync_copy(bins.at[pl.ds(q * L, L)],
                        cnt_sp.at[plsc.Indices(iden.at[q])], add=True)
    plsc.subcore_barrier()

    s0 = sid * (VT // NS)
    for z in range(VT // NS // 16):
        pltpu.sync_copy(cnt_sp.at[pl.ds(s0 + z * 16, 16)], invb)
        for rr in range(16):
            for k in range(8):
                cv = invb[rr, pl.ds(k * 16, 16)]
                outb[rr, pl.ds(k * 16, 16)] = 1.0 / jnp.maximum(cv, 1.0)
        pltpu.sync_copy(outb, inv_sp.at[pl.ds(s0 + z * 16, 16)])
    plsc.subcore_barrier()

    for c2 in range(2):
        ch = wid * 2 + c2
        g = ch // 8
        r = ch % 8
        zero_bins()

        def sbody(gi, carry):
            for b2 in range(2):
                j = gi * 2 + b2
                pltpu.make_async_copy(
                    f4d.at[g, pl.ds(j * 16, 16), r], fuseb.at[b2],
                    ssem).wait()
                pltpu.sync_copy(idx_sp.at[pl.ds(j * 16, 16)], idxb)

                @plsc.parallel_loop(0, 16, unroll=4)
                def _(rr):
                    for k in range(8):
                        iv = idxb[rr, pl.ds(k * 16, 16)]
                        hi = lax.shift_right_logical(iv, 7)
                        lo = lax.bitwise_and(iv, 127)
                        plsc.addupdate_scatter(
                            bins, [hi, lo], fuseb[b2, rr, pl.ds(k * 16, 16)])

                @pl.when(j + 2 < BT // 16)
                def _():
                    pltpu.async_copy(
                        f4d.at[g, pl.ds((j + 2) * 16, 16), r],
                        fuseb.at[b2], ssem)
            return carry

        for b2 in range(2):
            pltpu.async_copy(f4d.at[g, pl.ds(b2 * 16, 16), r],
                             fuseb.at[b2], ssem)
        lax.fori_loop(0, (BT // 16) // 2, sbody, 0)

        def wbody(w, carry):
            pltpu.sync_copy(inv_sp.at[pl.ds(w * 16, 16)], invb)

            @plsc.parallel_loop(0, 16, unroll=4)
            def _(rr):
                for k in range(8):
                    outb[rr, pl.ds(k * 16, 16)] = (
                        bins[w * 16 + rr, pl.ds(k * 16, 16)]
                        * invb[rr, pl.ds(k * 16, 16)])

            pltpu.sync_copy(outb, out4d.at[g, pl.ds(w * 16, 16), r])
            return carry

        lax.fori_loop(0, VT // 16, wbody, 0)


def _scatter_cm(f4d, idx2d):
    k = pl.kernel(
        _scatter_body,
        out_type=jax.ShapeDtypeStruct((CG, VT, 8, L), jnp.float32),
        mesh=_mesh,
        scratch_types=[
            pltpu.VMEM_SHARED((BT, L), jnp.int32),
            pltpu.VMEM_SHARED((VT, L), jnp.float32),
            pltpu.VMEM_SHARED((VT, L), jnp.float32),
            pltpu.VMEM_SHARED((16, L), jnp.float32),
            pltpu.VMEM((VT, L), jnp.float32),
            pltpu.VMEM((16, L), jnp.int32),
            pltpu.VMEM((2, 16, L), jnp.float32),
            pltpu.VMEM((16, L), jnp.float32),
            pltpu.VMEM((16, L), jnp.float32),
            pltpu.VMEM((16, L), jnp.float32),
            pltpu.VMEM((4, L), jnp.int32),
            pltpu.SemaphoreType.DMA,
        ],
        compiler_params=_sc_params,
    )
    return k(f4d, idx2d)


def kernel(p_feats, v_feats, p2v_idx, Wp, bp, Wv, bv):
    idx2d = p2v_idx.reshape(BT, L)
    wpd = jnp.broadcast_to((Wp[0] - Wp[1]).reshape(C, 1), (C, 128))
    wvd = jnp.broadcast_to((Wv[0] - Wv[1]).reshape(C, 1), (C, 128))
    bias = (bp[0] - bp[1] + bv[0] - bv[1]).reshape(1, 1)

    v4d = v_feats.reshape(VT, L, CG, 8).transpose(2, 0, 3, 1)
    pT = p_feats.T

    v2p4d = _gather_cm(v4d, idx2d)
    v2pT = v2p4d.transpose(0, 2, 1, 3).reshape(C, B)
    fuseT = _fuse_tc(pT, v2pT, wpd, wvd, bias)
    f4d = fuseT.reshape(CG, 8, BT, L).transpose(0, 2, 1, 3)
    vnew4d = _scatter_cm(f4d, idx2d)
    v_new = vnew4d.transpose(0, 2, 1, 3).reshape(C, V).T
    return fuseT.T, v_new


# speedup vs baseline: 7.6408x; 1.3547x over previous
import functools

import jax
import jax.numpy as jnp
from jax import lax
from jax.experimental import pallas as pl
from jax.experimental.pallas import tpu as pltpu
from jax.experimental.pallas import tpu_sc as plsc

B = 262144
V = 65536
C = 64
NC = 2
NS = 16
NW = NC * NS
L = 128
BT = B // L
VT = V // L
CG = C // 8

_mesh = plsc.VectorSubcoreMesh(core_axis_name="c", subcore_axis_name="s")
_sc_params = pltpu.CompilerParams(use_tc_tiling_on_sc=False,
                                  needs_layout_passes=False)


def _stage_idx(idx_hbm, idx_sp, sid):
    pltpu.sync_copy(idx_hbm.at[pl.ds(sid * (BT // NS), BT // NS)],
                    idx_sp.at[pl.ds(sid * (BT // NS), BT // NS)])
    plsc.subcore_barrier()


def _gather_body(v4d, idx_hbm, out4d, idx_sp, tab_v, idxb, outb, gsem, isem):
    cid = lax.axis_index("c")
    sid = lax.axis_index("s")
    wid = sid * NC + cid
    _stage_idx(idx_hbm, idx_sp, sid)

    GR = 64
    NG = BT // GR
    for c2 in range(2):
        ch = wid * 2 + c2
        g = ch // 8
        r = ch % 8
        pltpu.sync_copy(v4d.at[g, pl.ds(0, VT), r], tab_v)

        for b in range(2):
            pltpu.async_copy(idx_sp.at[pl.ds(b * GR, GR)], idxb.at[b], isem)

        def body(go, carry):
            for b in range(2):
                gi = go * 2 + b
                pltpu.make_async_copy(
                    idx_sp.at[pl.ds(gi * GR, GR)], idxb.at[b], isem).wait()

                @pl.when(gi >= 2)
                def _():
                    pltpu.make_async_copy(
                        outb.at[b],
                        out4d.at[g, pl.ds((gi - 2) * GR, GR), r],
                        gsem).wait()

                @plsc.parallel_loop(0, GR, unroll=4)
                def _(rr):
                    for k in range(8):
                        iv = idxb[b, rr, pl.ds(k * 16, 16)]
                        hi = lax.shift_right_logical(iv, 7)
                        lo = lax.bitwise_and(iv, 127)
                        outb[b, rr, pl.ds(k * 16, 16)] = (
                            plsc.load_gather(tab_v, [hi, lo]))

                pltpu.async_copy(outb.at[b],
                                 out4d.at[g, pl.ds(gi * GR, GR), r], gsem)

                @pl.when(gi + 2 < NG)
                def _():
                    pltpu.async_copy(idx_sp.at[pl.ds((gi + 2) * GR, GR)],
                                     idxb.at[b], isem)
            return carry

        lax.fori_loop(0, NG // 2, body, 0)
        for b in range(2):
            pltpu.make_async_copy(
                outb.at[b], out4d.at[g, pl.ds(0, GR), r], gsem).wait()


def _gather_cm(v4d, idx2d):
    k = pl.kernel(
        _gather_body,
        out_type=jax.ShapeDtypeStruct((CG, BT, 8, L), jnp.float32),
        mesh=_mesh,
        scratch_types=[
            pltpu.VMEM_SHARED((BT, L), jnp.int32),
            pltpu.VMEM((VT, L), jnp.float32),
            pltpu.VMEM((2, 64, L), jnp.int32),
            pltpu.VMEM((2, 64, L), jnp.float32),
            pltpu.SemaphoreType.DMA,
            pltpu.SemaphoreType.DMA,
        ],
        compiler_params=_sc_params,
    )
    return k(v4d, idx2d)


_BLKP = 4096


def _fuse_body(p_ref, v_ref, wpd_ref, wvd_ref, bias_ref, fuse_ref):
    p = p_ref[...]
    v = v_ref[...]
    wpd = wpd_ref[...][:, :1]
    wvd = wvd_ref[...][:, :1]
    d = (jnp.sum(p * wpd, axis=0, keepdims=True)
         + jnp.sum(v * wvd, axis=0, keepdims=True)
         + bias_ref[0, 0])
    s = 1.0 / (1.0 + jnp.exp(-d))
    fuse_ref[...] = v + s * (p - v)


def _fuse_tc(pT, v2pT, wpd, wvd, bias):
    return pl.pallas_call(
        _fuse_body,
        grid=(B // _BLKP,),
        in_specs=[
            pl.BlockSpec((C, _BLKP), lambda i: (0, i)),
            pl.BlockSpec((C, _BLKP), lambda i: (0, i)),
            pl.BlockSpec((C, 128), lambda i: (0, 0)),
            pl.BlockSpec((C, 128), lambda i: (0, 0)),
            pl.BlockSpec((1, 1), lambda i: (0, 0)),
        ],
        out_specs=pl.BlockSpec((C, _BLKP), lambda i: (0, i)),
        out_shape=jax.ShapeDtypeStruct((C, B), jnp.float32),
    )(pT, v2pT, wpd, wvd, bias)


def _scatter_body(f4d, idx_hbm, out4d,
                  idx_sp, cnt_sp, inv_sp, zsp,
                  bins, idxb, fuseb, invb, outb, zbuf, iden, ssem, isem):
    cid = lax.axis_index("c")
    sid = lax.axis_index("s")
    wid = sid * NC + cid
    zero16 = jnp.zeros((16,), jnp.float32)
    one16 = jnp.full((16,), 1.0, jnp.float32)

    for rr in range(16):
        for k in range(8):
            zbuf[rr, pl.ds(k * 16, 16)] = zero16
    for q in range(4):
        for k in range(8):
            iden[q, pl.ds(k * 16, 16)] = (
                lax.iota(jnp.int32, 16) + (q * L + k * 16))

    @pl.when(sid == 0)
    def _():
        pltpu.sync_copy(zbuf, zsp)

    _stage_idx(idx_hbm, idx_sp, sid)

    def zero_bins():
        for z in range(VT // 16):
            pltpu.sync_copy(zsp, bins.at[pl.ds(z * 16, 16)])

    zero_bins()
    for z in range(VT // NS // 16):
        pltpu.sync_copy(zbuf, cnt_sp.at[pl.ds(sid * (VT // NS) + z * 16, 16)])
    plsc.subcore_barrier()

    def cbody(j, carry):
        pltpu.sync_copy(idx_sp.at[pl.ds(sid * (BT // NS) + j * 4, 4)],
                        idxb.at[0, pl.ds(0, 4)])

        @plsc.parallel_loop(0, 4, unroll=2)
        def _(rr):
            for k in range(8):
                iv = idxb[0, rr, pl.ds(k * 16, 16)]
                hi = lax.shift_right_logical(iv, 7)
                lo = lax.bitwise_and(iv, 127)
                plsc.addupdate_scatter(bins, [hi, lo], one16)

        return carry

    lax.fori_loop(0, BT // NS // 4, cbody, 0)
    for q in range(4):
        pltpu.sync_copy(bins.at[pl.ds(q * L, L)],
                        cnt_sp.at[plsc.Indices(iden.at[q])], add=True)
    plsc.subcore_barrier()

    s0 = sid * (VT // NS)
    for z in range(VT // NS // 16):
        pltpu.sync_copy(cnt_sp.at[pl.ds(s0 + z * 16, 16)], invb)
        for rr in range(16):
            for k in range(8):
                cv = invb[rr, pl.ds(k * 16, 16)]
                outb[rr, pl.ds(k * 16, 16)] = 1.0 / jnp.maximum(cv, 1.0)
        pltpu.sync_copy(outb, inv_sp.at[pl.ds(s0 + z * 16, 16)])
    plsc.subcore_barrier()

    for c2 in range(2):
        ch = wid * 2 + c2
        g = ch // 8
        r = ch % 8
        zero_bins()

        GR = 64
        NG = BT // GR

        def sbody(go, carry):
            for b2 in range(2):
                gi = go * 2 + b2
                pltpu.make_async_copy(
                    f4d.at[g, pl.ds(gi * GR, GR), r], fuseb.at[b2],
                    ssem).wait()
                pltpu.make_async_copy(
                    idx_sp.at[pl.ds(gi * GR, GR)], idxb.at[b2], isem).wait()

                @plsc.parallel_loop(0, GR, unroll=4)
                def _(rr):
                    for k in range(8):
                        iv = idxb[b2, rr, pl.ds(k * 16, 16)]
                        hi = lax.shift_right_logical(iv, 7)
                        lo = lax.bitwise_and(iv, 127)
                        plsc.addupdate_scatter(
                            bins, [hi, lo], fuseb[b2, rr, pl.ds(k * 16, 16)])

                @pl.when(gi + 2 < NG)
                def _():
                    pltpu.async_copy(
                        f4d.at[g, pl.ds((gi + 2) * GR, GR), r],
                        fuseb.at[b2], ssem)
                    pltpu.async_copy(idx_sp.at[pl.ds((gi + 2) * GR, GR)],
                                     idxb.at[b2], isem)
            return carry

        for b2 in range(2):
            pltpu.async_copy(f4d.at[g, pl.ds(b2 * GR, GR), r],
                             fuseb.at[b2], ssem)
            pltpu.async_copy(idx_sp.at[pl.ds(b2 * GR, GR)],
                             idxb.at[b2], isem)
        lax.fori_loop(0, NG // 2, sbody, 0)

        def wbody(w, carry):
            pltpu.sync_copy(inv_sp.at[pl.ds(w * 16, 16)], invb)

            @plsc.parallel_loop(0, 16, unroll=4)
            def _(rr):
                for k in range(8):
                    outb[rr, pl.ds(k * 16, 16)] = (
                        bins[w * 16 + rr, pl.ds(k * 16, 16)]
                        * invb[rr, pl.ds(k * 16, 16)])

            pltpu.sync_copy(outb, out4d.at[g, pl.ds(w * 16, 16), r])
            return carry

        lax.fori_loop(0, VT // 16, wbody, 0)


def _scatter_cm(f4d, idx2d):
    k = pl.kernel(
        _scatter_body,
        out_type=jax.ShapeDtypeStruct((CG, VT, 8, L), jnp.float32),
        mesh=_mesh,
        scratch_types=[
            pltpu.VMEM_SHARED((BT, L), jnp.int32),
            pltpu.VMEM_SHARED((VT, L), jnp.float32),
            pltpu.VMEM_SHARED((VT, L), jnp.float32),
            pltpu.VMEM_SHARED((16, L), jnp.float32),
            pltpu.VMEM((VT, L), jnp.float32),
            pltpu.VMEM((2, 64, L), jnp.int32),
            pltpu.VMEM((2, 64, L), jnp.float32),
            pltpu.VMEM((16, L), jnp.float32),
            pltpu.VMEM((16, L), jnp.float32),
            pltpu.VMEM((16, L), jnp.float32),
            pltpu.VMEM((4, L), jnp.int32),
            pltpu.SemaphoreType.DMA,
            pltpu.SemaphoreType.DMA,
        ],
        compiler_params=_sc_params,
    )
    return k(f4d, idx2d)


def kernel(p_feats, v_feats, p2v_idx, Wp, bp, Wv, bv):
    idx2d = p2v_idx.reshape(BT, L)
    wpd = jnp.broadcast_to((Wp[0] - Wp[1]).reshape(C, 1), (C, 128))
    wvd = jnp.broadcast_to((Wv[0] - Wv[1]).reshape(C, 1), (C, 128))
    bias = (bp[0] - bp[1] + bv[0] - bv[1]).reshape(1, 1)

    v4d = v_feats.reshape(VT, L, CG, 8).transpose(2, 0, 3, 1)
    pT = p_feats.T

    v2p4d = _gather_cm(v4d, idx2d)
    v2pT = v2p4d.transpose(0, 2, 1, 3).reshape(C, B)
    fuseT = _fuse_tc(pT, v2pT, wpd, wvd, bias)
    f4d = fuseT.reshape(CG, 8, BT, L).transpose(0, 2, 1, 3)
    vnew4d = _scatter_cm(f4d, idx2d)
    v_new = vnew4d.transpose(0, 2, 1, 3).reshape(C, V).T
    return fuseT.T, v_new


# speedup vs baseline: 8.0648x; 1.0555x over previous
import functools

import jax
import jax.numpy as jnp
from jax import lax
from jax.experimental import pallas as pl
from jax.experimental.pallas import tpu as pltpu
from jax.experimental.pallas import tpu_sc as plsc

B = 262144
V = 65536
C = 64
NC = 2
NS = 16
NW = NC * NS
L = 128
BT = B // L
VT = V // L
CG = C // 8

_mesh = plsc.VectorSubcoreMesh(core_axis_name="c", subcore_axis_name="s")
_sc_params = pltpu.CompilerParams(use_tc_tiling_on_sc=False,
                                  needs_layout_passes=False)


def _stage_idx(idx_hbm, idx_sp, sid):
    pltpu.sync_copy(idx_hbm.at[pl.ds(sid * (BT // NS), BT // NS)],
                    idx_sp.at[pl.ds(sid * (BT // NS), BT // NS)])
    plsc.subcore_barrier()


def _gather_body(v4d, idx_hbm, out4d, idx_sp, tab_v, idxb, outb, gsem, isem):
    cid = lax.axis_index("c")
    sid = lax.axis_index("s")
    wid = sid * NC + cid
    _stage_idx(idx_hbm, idx_sp, sid)

    GR = 64
    NG = BT // GR
    for c2 in range(2):
        ch = wid * 2 + c2
        g = ch // 8
        r = ch % 8
        pltpu.sync_copy(v4d.at[g, pl.ds(0, VT), r], tab_v)

        for b in range(2):
            pltpu.async_copy(idx_sp.at[pl.ds(b * GR, GR)], idxb.at[b], isem)

        def body(go, carry):
            for b in range(2):
                gi = go * 2 + b
                pltpu.make_async_copy(
                    idx_sp.at[pl.ds(gi * GR, GR)], idxb.at[b], isem).wait()

                @pl.when(gi >= 2)
                def _():
                    pltpu.make_async_copy(
                        outb.at[b],
                        out4d.at[g, pl.ds((gi - 2) * GR, GR), r],
                        gsem).wait()

                @plsc.parallel_loop(0, GR, unroll=4)
                def _(rr):
                    for k in range(8):
                        iv = idxb[b, rr, pl.ds(k * 16, 16)]
                        hi = lax.shift_right_logical(iv, 7)
                        lo = lax.bitwise_and(iv, 127)
                        outb[b, rr, pl.ds(k * 16, 16)] = (
                            plsc.load_gather(tab_v, [hi, lo]))

                pltpu.async_copy(outb.at[b],
                                 out4d.at[g, pl.ds(gi * GR, GR), r], gsem)

                @pl.when(gi + 2 < NG)
                def _():
                    pltpu.async_copy(idx_sp.at[pl.ds((gi + 2) * GR, GR)],
                                     idxb.at[b], isem)
            return carry

        lax.fori_loop(0, NG // 2, body, 0)
        for b in range(2):
            pltpu.make_async_copy(
                outb.at[b], out4d.at[g, pl.ds(0, GR), r], gsem).wait()


def _gather_cm(v4d, idx2d):
    k = pl.kernel(
        _gather_body,
        out_type=jax.ShapeDtypeStruct((CG, BT, 8, L), jnp.float32),
        mesh=_mesh,
        scratch_types=[
            pltpu.VMEM_SHARED((BT, L), jnp.int32),
            pltpu.VMEM((VT, L), jnp.float32),
            pltpu.VMEM((2, 64, L), jnp.int32),
            pltpu.VMEM((2, 64, L), jnp.float32),
            pltpu.SemaphoreType.DMA,
            pltpu.SemaphoreType.DMA,
        ],
        compiler_params=_sc_params,
    )
    return k(v4d, idx2d)


_BLKP = 8192


def _fuse_body(p_ref, v_ref, wpd_ref, wvd_ref, bias_ref, fuse_ref):
    p = p_ref[...]
    v = v_ref[...]
    wpd = wpd_ref[...][:, :1]
    wvd = wvd_ref[...][:, :1]
    d = (jnp.sum(p * wpd, axis=0, keepdims=True)
         + jnp.sum(v * wvd, axis=0, keepdims=True)
         + bias_ref[0, 0])
    s = 1.0 / (1.0 + jnp.exp(-d))
    fuse_ref[...] = v + s * (p - v)


def _fuse_tc(pT, v2pT, wpd, wvd, bias):
    return pl.pallas_call(
        _fuse_body,
        grid=(B // _BLKP,),
        in_specs=[
            pl.BlockSpec((C, _BLKP), lambda i: (0, i)),
            pl.BlockSpec((C, _BLKP), lambda i: (0, i)),
            pl.BlockSpec((C, 128), lambda i: (0, 0)),
            pl.BlockSpec((C, 128), lambda i: (0, 0)),
            pl.BlockSpec((1, 1), lambda i: (0, 0)),
        ],
        out_specs=pl.BlockSpec((C, _BLKP), lambda i: (0, i)),
        out_shape=jax.ShapeDtypeStruct((C, B), jnp.float32),
    )(pT, v2pT, wpd, wvd, bias)


def _scatter_body(f4d, idx_hbm, out4d,
                  idx_sp, cnt_sp, inv_sp, zsp,
                  bins, idxb, fuseb, invb, outb, zbuf, iden, ssem, isem):
    cid = lax.axis_index("c")
    sid = lax.axis_index("s")
    wid = sid * NC + cid
    zero16 = jnp.zeros((16,), jnp.float32)
    one16 = jnp.full((16,), 1.0, jnp.float32)

    for rr in range(16):
        for k in range(8):
            zbuf[rr, pl.ds(k * 16, 16)] = zero16
    for q in range(4):
        for k in range(8):
            iden[q, pl.ds(k * 16, 16)] = (
                lax.iota(jnp.int32, 16) + (q * L + k * 16))

    @pl.when(sid == 0)
    def _():
        pltpu.sync_copy(zbuf, zsp)

    _stage_idx(idx_hbm, idx_sp, sid)

    def zero_bins():
        for z in range(VT // 16):
            pltpu.sync_copy(zsp, bins.at[pl.ds(z * 16, 16)])

    zero_bins()
    for z in range(VT // NS // 16):
        pltpu.sync_copy(zbuf, cnt_sp.at[pl.ds(sid * (VT // NS) + z * 16, 16)])
    plsc.subcore_barrier()

    def cbody(j, carry):
        pltpu.sync_copy(idx_sp.at[pl.ds(sid * (BT // NS) + j * 4, 4)],
                        idxb.at[0, pl.ds(0, 4)])

        @plsc.parallel_loop(0, 4, unroll=2)
        def _(rr):
            for k in range(8):
                iv = idxb[0, rr, pl.ds(k * 16, 16)]
                hi = lax.shift_right_logical(iv, 7)
                lo = lax.bitwise_and(iv, 127)
                plsc.addupdate_scatter(bins, [hi, lo], one16)

        return carry

    lax.fori_loop(0, BT // NS // 4, cbody, 0)
    for q in range(4):
        pltpu.sync_copy(bins.at[pl.ds(q * L, L)],
                        cnt_sp.at[plsc.Indices(iden.at[q])], add=True)
    plsc.subcore_barrier()

    s0 = sid * (VT // NS)
    for z in range(VT // NS // 16):
        pltpu.sync_copy(cnt_sp.at[pl.ds(s0 + z * 16, 16)], invb)
        for rr in range(16):
            for k in range(8):
                cv = invb[rr, pl.ds(k * 16, 16)]
                outb[rr, pl.ds(k * 16, 16)] = 1.0 / jnp.maximum(cv, 1.0)
        pltpu.sync_copy(outb, inv_sp.at[pl.ds(s0 + z * 16, 16)])
    plsc.subcore_barrier()

    for c2 in range(2):
        ch = wid * 2 + c2
        g = ch // 8
        r = ch % 8
        zero_bins()

        GR = 64
        NG = BT // GR

        def sbody(go, carry):
            for b2 in range(2):
                gi = go * 2 + b2
                pltpu.make_async_copy(
                    f4d.at[g, pl.ds(gi * GR, GR), r], fuseb.at[b2],
                    ssem).wait()
                pltpu.make_async_copy(
                    idx_sp.at[pl.ds(gi * GR, GR)], idxb.at[b2], isem).wait()

                @plsc.parallel_loop(0, GR, unroll=4)
                def _(rr):
                    for k in range(8):
                        iv = idxb[b2, rr, pl.ds(k * 16, 16)]
                        hi = lax.shift_right_logical(iv, 7)
                        lo = lax.bitwise_and(iv, 127)
                        plsc.addupdate_scatter(
                            bins, [hi, lo], fuseb[b2, rr, pl.ds(k * 16, 16)])

                @pl.when(gi + 2 < NG)
                def _():
                    pltpu.async_copy(
                        f4d.at[g, pl.ds((gi + 2) * GR, GR), r],
                        fuseb.at[b2], ssem)
                    pltpu.async_copy(idx_sp.at[pl.ds((gi + 2) * GR, GR)],
                                     idxb.at[b2], isem)
            return carry

        for b2 in range(2):
            pltpu.async_copy(f4d.at[g, pl.ds(b2 * GR, GR), r],
                             fuseb.at[b2], ssem)
            pltpu.async_copy(idx_sp.at[pl.ds(b2 * GR, GR)],
                             idxb.at[b2], isem)
        lax.fori_loop(0, NG // 2, sbody, 0)

        def wbody(w, carry):
            pltpu.sync_copy(inv_sp.at[pl.ds(w * 16, 16)], invb)

            @plsc.parallel_loop(0, 16, unroll=4)
            def _(rr):
                for k in range(8):
                    outb[rr, pl.ds(k * 16, 16)] = (
                        bins[w * 16 + rr, pl.ds(k * 16, 16)]
                        * invb[rr, pl.ds(k * 16, 16)])

            pltpu.sync_copy(outb, out4d.at[g, pl.ds(w * 16, 16), r])
            return carry

        lax.fori_loop(0, VT // 16, wbody, 0)


def _scatter_cm(f4d, idx2d):
    k = pl.kernel(
        _scatter_body,
        out_type=jax.ShapeDtypeStruct((CG, VT, 8, L), jnp.float32),
        mesh=_mesh,
        scratch_types=[
            pltpu.VMEM_SHARED((BT, L), jnp.int32),
            pltpu.VMEM_SHARED((VT, L), jnp.float32),
            pltpu.VMEM_SHARED((VT, L), jnp.float32),
            pltpu.VMEM_SHARED((16, L), jnp.float32),
            pltpu.VMEM((VT, L), jnp.float32),
            pltpu.VMEM((2, 64, L), jnp.int32),
            pltpu.VMEM((2, 64, L), jnp.float32),
            pltpu.VMEM((16, L), jnp.float32),
            pltpu.VMEM((16, L), jnp.float32),
            pltpu.VMEM((16, L), jnp.float32),
            pltpu.VMEM((4, L), jnp.int32),
            pltpu.SemaphoreType.DMA,
            pltpu.SemaphoreType.DMA,
        ],
        compiler_params=_sc_params,
    )
    return k(f4d, idx2d)


def kernel(p_feats, v_feats, p2v_idx, Wp, bp, Wv, bv):
    idx2d = p2v_idx.reshape(BT, L)
    wpd = jnp.broadcast_to((Wp[0] - Wp[1]).reshape(C, 1), (C, 128))
    wvd = jnp.broadcast_to((Wv[0] - Wv[1]).reshape(C, 1), (C, 128))
    bias = (bp[0] - bp[1] + bv[0] - bv[1]).reshape(1, 1)

    v4d = v_feats.reshape(VT, L, CG, 8).transpose(2, 0, 3, 1)
    pT = p_feats.T

    v2p4d = _gather_cm(v4d, idx2d)
    v2pT = v2p4d.transpose(0, 2, 1, 3).reshape(C, B)
    fuseT = _fuse_tc(pT, v2pT, wpd, wvd, bias)
    f4d = fuseT.reshape(CG, 8, BT, L).transpose(0, 2, 1, 3)
    vnew4d = _scatter_cm(f4d, idx2d)
    v_new = vnew4d.transpose(0, 2, 1, 3).reshape(C, V).T
    return fuseT.T, v_new


# speedup vs baseline: 8.0763x; 1.0014x over previous
import jax
import jax.numpy as jnp
from jax import lax
from jax.experimental import pallas as pl
from jax.experimental.pallas import tpu as pltpu
from jax.experimental.pallas import tpu_sc as plsc

B = 262144
V = 65536
C = 64
NC = 2
NS = 16
NW = NC * NS
L = 128
BT = B // L
VT = V // L
CG = C // 8

_mesh = plsc.VectorSubcoreMesh(core_axis_name="c", subcore_axis_name="s")
_sc_params = pltpu.CompilerParams(use_tc_tiling_on_sc=False,
                                  needs_layout_passes=False)


def _stage_idx(idx_hbm, idx_sp, sid):
    pltpu.sync_copy(idx_hbm.at[pl.ds(sid * (BT // NS), BT // NS)],
                    idx_sp.at[pl.ds(sid * (BT // NS), BT // NS)])
    plsc.subcore_barrier()


def _gather_body(v4d, idx_hbm, out4d, idx_sp, tab_v, idxb, outb, gsem, isem):
    cid = lax.axis_index("c")
    sid = lax.axis_index("s")
    wid = sid * NC + cid
    _stage_idx(idx_hbm, idx_sp, sid)

    GR = 64
    NG = BT // GR
    for c2 in range(2):
        ch = wid * 2 + c2
        g = ch // 8
        r = ch % 8
        pltpu.sync_copy(v4d.at[g, pl.ds(0, VT), r], tab_v)

        for b in range(2):
            pltpu.async_copy(idx_sp.at[pl.ds(b * GR, GR)], idxb.at[b], isem)

        def body(go, carry):
            for b in range(2):
                gi = go * 2 + b
                pltpu.make_async_copy(
                    idx_sp.at[pl.ds(gi * GR, GR)], idxb.at[b], isem).wait()

                @pl.when(gi >= 2)
                def _():
                    pltpu.make_async_copy(
                        outb.at[b],
                        out4d.at[g, pl.ds((gi - 2) * GR, GR), r],
                        gsem).wait()

                @plsc.parallel_loop(0, GR, unroll=4)
                def _(rr):
                    for k in range(8):
                        iv = idxb[b, rr, pl.ds(k * 16, 16)]
                        hi = lax.shift_right_logical(iv, 7)
                        lo = lax.bitwise_and(iv, 127)
                        outb[b, rr, pl.ds(k * 16, 16)] = (
                            plsc.load_gather(tab_v, [hi, lo]))

                pltpu.async_copy(outb.at[b],
                                 out4d.at[g, pl.ds(gi * GR, GR), r], gsem)

                @pl.when(gi + 2 < NG)
                def _():
                    pltpu.async_copy(idx_sp.at[pl.ds((gi + 2) * GR, GR)],
                                     idxb.at[b], isem)
            return carry

        lax.fori_loop(0, NG // 2, body, 0)
        for b in range(2):
            pltpu.make_async_copy(
                outb.at[b], out4d.at[g, pl.ds(0, GR), r], gsem).wait()


def _gather_cm(v4d, idx2d):
    k = pl.kernel(
        _gather_body,
        out_type=jax.ShapeDtypeStruct((CG, BT, 8, L), jnp.float32),
        mesh=_mesh,
        scratch_types=[
            pltpu.VMEM_SHARED((BT, L), jnp.int32),
            pltpu.VMEM((VT, L), jnp.float32),
            pltpu.VMEM((2, 64, L), jnp.int32),
            pltpu.VMEM((2, 64, L), jnp.float32),
            pltpu.SemaphoreType.DMA,
            pltpu.SemaphoreType.DMA,
        ],
        compiler_params=_sc_params,
    )
    return k(v4d, idx2d)


_BLKP = 8192


def _fuse_body(p_ref, v_ref, wpd_ref, wvd_ref, bias_ref, fuse_ref):
    p = p_ref[...]
    v = v_ref[...]
    wpd = wpd_ref[...][:, :1]
    wvd = wvd_ref[...][:, :1]
    d = (jnp.sum(p * wpd, axis=0, keepdims=True)
         + jnp.sum(v * wvd, axis=0, keepdims=True)
         + bias_ref[0, 0])
    s = 1.0 / (1.0 + jnp.exp(-d))
    fuse_ref[...] = v + s * (p - v)


def _fuse_tc(pT, v2pT, wpd, wvd, bias):
    return pl.pallas_call(
        _fuse_body,
        grid=(B // _BLKP,),
        in_specs=[
            pl.BlockSpec((C, _BLKP), lambda i: (0, i)),
            pl.BlockSpec((C, _BLKP), lambda i: (0, i)),
            pl.BlockSpec((C, 128), lambda i: (0, 0)),
            pl.BlockSpec((C, 128), lambda i: (0, 0)),
            pl.BlockSpec((1, 1), lambda i: (0, 0)),
        ],
        out_specs=pl.BlockSpec((C, _BLKP), lambda i: (0, i)),
        out_shape=jax.ShapeDtypeStruct((C, B), jnp.float32),
    )(pT, v2pT, wpd, wvd, bias)


def _scatter_body(f4d, idx_hbm, out4d,
                  idx_sp, cnt_sp, inv_sp, zsp,
                  bins, idxb, fuseb, invb, outb, zbuf, iden, ssem, isem):
    cid = lax.axis_index("c")
    sid = lax.axis_index("s")
    wid = sid * NC + cid
    zero16 = jnp.zeros((16,), jnp.float32)
    one16 = jnp.full((16,), 1.0, jnp.float32)

    for rr in range(16):
        for k in range(8):
            zbuf[rr, pl.ds(k * 16, 16)] = zero16
    for q in range(4):
        for k in range(8):
            iden[q, pl.ds(k * 16, 16)] = (
                lax.iota(jnp.int32, 16) + (q * L + k * 16))

    @pl.when(sid == 0)
    def _():
        pltpu.sync_copy(zbuf, zsp)

    _stage_idx(idx_hbm, idx_sp, sid)

    def zero_bins():
        for z in range(VT // 16):
            pltpu.sync_copy(zsp, bins.at[pl.ds(z * 16, 16)])

    zero_bins()
    for z in range(VT // NS // 16):
        pltpu.sync_copy(zbuf, cnt_sp.at[pl.ds(sid * (VT // NS) + z * 16, 16)])
    plsc.subcore_barrier()

    def cbody(j, carry):
        pltpu.sync_copy(idx_sp.at[pl.ds(sid * (BT // NS) + j * 4, 4)],
                        idxb.at[0, pl.ds(0, 4)])

        @plsc.parallel_loop(0, 4, unroll=2)
        def _(rr):
            for k in range(8):
                iv = idxb[0, rr, pl.ds(k * 16, 16)]
                hi = lax.shift_right_logical(iv, 7)
                lo = lax.bitwise_and(iv, 127)
                plsc.addupdate_scatter(bins, [hi, lo], one16)

        return carry

    lax.fori_loop(0, BT // NS // 4, cbody, 0)
    for q in range(4):
        pltpu.sync_copy(bins.at[pl.ds(q * L, L)],
                        cnt_sp.at[plsc.Indices(iden.at[q])], add=True)
    plsc.subcore_barrier()

    s0 = sid * (VT // NS)
    for z in range(VT // NS // 16):
        pltpu.sync_copy(cnt_sp.at[pl.ds(s0 + z * 16, 16)], invb)
        for rr in range(16):
            for k in range(8):
                cv = invb[rr, pl.ds(k * 16, 16)]
                outb[rr, pl.ds(k * 16, 16)] = 1.0 / jnp.maximum(cv, 1.0)
        pltpu.sync_copy(outb, inv_sp.at[pl.ds(s0 + z * 16, 16)])
    plsc.subcore_barrier()

    for c2 in range(2):
        ch = wid * 2 + c2
        g = ch // 8
        r = ch % 8
        zero_bins()

        GR = 64
        NG = BT // GR

        def sbody(go, carry):
            for b2 in range(2):
                gi = go * 2 + b2
                pltpu.make_async_copy(
                    f4d.at[g, pl.ds(gi * GR, GR), r], fuseb.at[b2],
                    ssem).wait()
                pltpu.make_async_copy(
                    idx_sp.at[pl.ds(gi * GR, GR)], idxb.at[b2], isem).wait()

                @plsc.parallel_loop(0, GR, unroll=4)
                def _(rr):
                    for k in range(8):
                        iv = idxb[b2, rr, pl.ds(k * 16, 16)]
                        hi = lax.shift_right_logical(iv, 7)
                        lo = lax.bitwise_and(iv, 127)
                        plsc.addupdate_scatter(
                            bins, [hi, lo], fuseb[b2, rr, pl.ds(k * 16, 16)])

                @pl.when(gi + 2 < NG)
                def _():
                    pltpu.async_copy(
                        f4d.at[g, pl.ds((gi + 2) * GR, GR), r],
                        fuseb.at[b2], ssem)
                    pltpu.async_copy(idx_sp.at[pl.ds((gi + 2) * GR, GR)],
                                     idxb.at[b2], isem)
            return carry

        for b2 in range(2):
            pltpu.async_copy(f4d.at[g, pl.ds(b2 * GR, GR), r],
                             fuseb.at[b2], ssem)
            pltpu.async_copy(idx_sp.at[pl.ds(b2 * GR, GR)],
                             idxb.at[b2], isem)
        lax.fori_loop(0, NG // 2, sbody, 0)

        def wbody(w, carry):
            pltpu.sync_copy(inv_sp.at[pl.ds(w * 16, 16)], invb)

            @plsc.parallel_loop(0, 16, unroll=4)
            def _(rr):
                for k in range(8):
                    outb[rr, pl.ds(k * 16, 16)] = (
                        bins[w * 16 + rr, pl.ds(k * 16, 16)]
                        * invb[rr, pl.ds(k * 16, 16)])

            pltpu.sync_copy(outb, out4d.at[g, pl.ds(w * 16, 16), r])
            return carry

        lax.fori_loop(0, VT // 16, wbody, 0)


def _scatter_cm(f4d, idx2d):
    k = pl.kernel(
        _scatter_body,
        out_type=jax.ShapeDtypeStruct((CG, VT, 8, L), jnp.float32),
        mesh=_mesh,
        scratch_types=[
            pltpu.VMEM_SHARED((BT, L), jnp.int32),
            pltpu.VMEM_SHARED((VT, L), jnp.float32),
            pltpu.VMEM_SHARED((VT, L), jnp.float32),
            pltpu.VMEM_SHARED((16, L), jnp.float32),
            pltpu.VMEM((VT, L), jnp.float32),
            pltpu.VMEM((2, 64, L), jnp.int32),
            pltpu.VMEM((2, 64, L), jnp.float32),
            pltpu.VMEM((16, L), jnp.float32),
            pltpu.VMEM((16, L), jnp.float32),
            pltpu.VMEM((16, L), jnp.float32),
            pltpu.VMEM((4, L), jnp.int32),
            pltpu.SemaphoreType.DMA,
            pltpu.SemaphoreType.DMA,
        ],
        compiler_params=_sc_params,
    )
    return k(f4d, idx2d)


def kernel(p_feats, v_feats, p2v_idx, Wp, bp, Wv, bv):
    idx2d = p2v_idx.reshape(BT, L)
    wpd = jnp.broadcast_to((Wp[0] - Wp[1]).reshape(C, 1), (C, 128))
    wvd = jnp.broadcast_to((Wv[0] - Wv[1]).reshape(C, 1), (C, 128))
    bias = (bp[0] - bp[1] + bv[0] - bv[1]).reshape(1, 1)

    v4d = v_feats.reshape(VT, L, CG, 8).transpose(2, 0, 3, 1)
    pT = p_feats.T

    v2p4d = _gather_cm(v4d, idx2d)
    v2pT = v2p4d.transpose(0, 2, 1, 3).reshape(C, B)
    fuseT = _fuse_tc(pT, v2pT, wpd, wvd, bias)
    f4d = fuseT.reshape(CG, 8, BT, L).transpose(0, 2, 1, 3)
    vnew4d = _scatter_cm(f4d, idx2d)
    v_new = vnew4d.transpose(0, 2, 1, 3).reshape(C, V).T
    return fuseT.T, v_new
